# baseline (speedup 1.0000x reference)
import functools

import jax
import jax.numpy as jnp
from jax import lax
from jax.experimental import pallas as pl
from jax.experimental.pallas import tpu as pltpu
from jax.experimental.pallas import tpu_sc as plsc

B = 16384
F = 26
V = 100000
D = 32
ND = 13
NF = F + ND
FC = 13

NC = 2
NS = 16
NW = NC * NS

RPF = B // 128


def _mm_body(xt_ref, wt_ref, o_ref):
    o_ref[...] = jnp.dot(wt_ref[...], xt_ref[...],
                         preferred_element_type=jnp.float32)


def _dense_proj_t(xt, wt):
    nb = 2048
    return pl.pallas_call(
        _mm_body,
        grid=(B // nb,),
        in_specs=[pl.BlockSpec((ND, nb), lambda i: (0, i)),
                  pl.BlockSpec((ND * D, ND), lambda i: (0, 0))],
        out_specs=pl.BlockSpec((ND * D, nb), lambda i: (0, i)),
        out_shape=jax.ShapeDtypeStruct((ND * D, B), jnp.float32),
    )(xt, wt)


def _make_sc_body(fc, with_dense):
    def _body(*refs):
        if with_dense:
            (tbl_hbm, sidx_hbm, dtok_hbm, out_hbm, dout_hbm,
             idxv, gbuf, dbuf, isem, gsemA, gsemB, ssem, dlsem, dsem) = refs
        else:
            (tbl_hbm, sidx_hbm, out_hbm,
             idxv, gbuf, isem, gsemA, gsemB, ssem) = refs
        wid = lax.axis_index("s") * NC + lax.axis_index("c")

        def fire_gathers(f, par, gsem):
            goff = par * B
            slab = tbl_hbm.at[f * D + wid]

            def gblk(r8, c):
                for j in range(8):
                    r = r8 * 8 + j
                    pltpu.async_copy(slab.at[idxv.at[par, r]],
                                     gbuf.at[pl.ds(goff + r * 128, 128)],
                                     gsem)
                return c
            lax.fori_loop(0, RPF // 8, gblk, 0)

        def drain_gathers(par, gsem):
            pltpu.make_async_copy(tbl_hbm.at[0].at[pl.ds(0, B)],
                                  gbuf.at[pl.ds(par * B, B)], gsem).wait()

        def drain_elems(sem):
            pltpu.make_async_copy(gbuf.at[pl.ds(0, B)],
                                  out_hbm.at[0], sem).wait()

        def drain_idx():
            pltpu.make_async_copy(sidx_hbm.at[0], idxv.at[0], isem).wait()

        pltpu.sync_copy(sidx_hbm.at[0], idxv.at[0])
        fire_gathers(0, 0, gsemA)
        pltpu.async_copy(sidx_hbm.at[1], idxv.at[1], isem)

        def field_body(f, carry):
            par = lax.rem(f, 2)

            drain_idx()

            @pl.when(f >= 2)
            def _():
                drain_elems(ssem)

            @pl.when(par == 0)
            def _():
                fire_gathers(f, 0, gsemA)
                drain_gathers(1, gsemB)

            @pl.when(par == 1)
            def _():
                fire_gathers(f, 1, gsemB)
                drain_gathers(0, gsemA)

            pltpu.async_copy(gbuf.at[pl.ds((1 - par) * B, B)],
                             out_hbm.at[(f - 1) * D + wid], ssem)

            @pl.when(f < fc - 1)
            def _():
                pltpu.async_copy(sidx_hbm.at[f + 1], idxv.at[1 - par], isem)

            if with_dense:
                jd = f - 2

                @pl.when(jd >= 0)
                def _():
                    @pl.when(jd >= 2)
                    def _():
                        drain_elems(dsem)
                    pltpu.async_copy(dtok_hbm.at[jd * D + wid],
                                     dbuf.at[pl.ds(lax.rem(jd, 2) * B, B)],
                                     dlsem)

                    @pl.when(jd >= 1)
                    def _():
                        drain_elems(dlsem)
                        pltpu.async_copy(
                            dbuf.at[pl.ds(lax.rem(jd - 1, 2) * B, B)],
                            dout_hbm.at[(jd - 1) * D + wid], dsem)
            return carry
        lax.fori_loop(1, fc, field_body, 0)

        lastpar = (fc - 1) % 2
        drain_gathers(lastpar, gsemB if lastpar else gsemA)
        pltpu.async_copy(gbuf.at[pl.ds(lastpar * B, B)],
                         out_hbm.at[(fc - 1) * D + wid], ssem)
        drain_elems(ssem)
        drain_elems(ssem)

        if with_dense:
            jmax = fc - 3
            drain_elems(dsem)
            drain_elems(dlsem)
            pltpu.async_copy(dbuf.at[pl.ds((jmax % 2) * B, B)],
                             dout_hbm.at[jmax * D + wid], dsem)
            drain_elems(dsem)
            for j in range(jmax + 1, ND):
                pltpu.async_copy(dtok_hbm.at[j * D + wid],
                                 dbuf.at[pl.ds((j % 2) * B, B)], dlsem)
                drain_elems(dlsem)
                pltpu.async_copy(dbuf.at[pl.ds((j % 2) * B, B)],
                                 dout_hbm.at[j * D + wid], dsem)
                drain_elems(dsem)
    return _body


@jax.jit
def _tokenize(sparse_inputs, dense_inputs, tables, w):
    tbl1 = tables[:FC].transpose(0, 2, 1).reshape(FC * D, V)
    tbl2 = tables[FC:].transpose(0, 2, 1).reshape((F - FC) * D, V)
    sidx_t = sparse_inputs.T.reshape(F, RPF, 128)
    dtok_t = _dense_proj_t(dense_inputs.T, w.T)

    mesh = plsc.VectorSubcoreMesh(core_axis_name="c", subcore_axis_name="s",
                                  num_cores=NC, num_subcores=NS)
    common = dict(mesh=mesh,
                  compiler_params=pltpu.CompilerParams(
                      use_tc_tiling_on_sc=False))
    sems = [pltpu.SemaphoreType.DMA] * 3
    sc1 = pl.kernel(
        _make_sc_body(FC, True),
        out_type=[jax.ShapeDtypeStruct((FC * D, B), jnp.float32),
                  jax.ShapeDtypeStruct((ND * D, B), jnp.float32)],
        scratch_types=[
            pltpu.VMEM((2, RPF, 128), jnp.int32),
            pltpu.VMEM((2 * B,), jnp.float32),
            pltpu.VMEM((2 * B,), jnp.float32),
        ] + [pltpu.SemaphoreType.DMA] * 6,
        **common)
    sc2 = pl.kernel(
        _make_sc_body(F - FC, False),
        out_type=jax.ShapeDtypeStruct(((F - FC) * D, B), jnp.float32),
        scratch_types=[
            pltpu.VMEM((2, RPF, 128), jnp.int32),
            pltpu.VMEM((2 * B,), jnp.float32),
        ] + [pltpu.SemaphoreType.DMA] * 4,
        **common)
    out1, dout = sc1(tbl1, sidx_t[:FC], dtok_t)
    out2 = sc2(tbl2, sidx_t[FC:])
    out = jnp.concatenate([out1, out2, dout], axis=0)
    return out.reshape(NF, D, B).transpose(2, 0, 1)


def kernel(sparse_inputs, dense_inputs, tables, W):
    return _tokenize(sparse_inputs, dense_inputs, tables, W)


# speedup vs baseline: 1.0962x; 1.0962x over previous
import functools

import jax
import jax.numpy as jnp
from jax import lax
from jax.experimental import pallas as pl
from jax.experimental.pallas import tpu as pltpu
from jax.experimental.pallas import tpu_sc as plsc

B = 16384
F = 26
V = 100000
D = 32
ND = 13
NF = F + ND

NC = 2
NS = 16
NW = NC * NS

RPF = B // 128


def _mm_body(xt_ref, wt_ref, o_ref):
    o_ref[...] = jnp.dot(wt_ref[...], xt_ref[...],
                         preferred_element_type=jnp.float32)


def _dense_proj_t(xt, wt):
    nb = 2048
    return pl.pallas_call(
        _mm_body,
        grid=(B // nb,),
        in_specs=[pl.BlockSpec((ND, nb), lambda i: (0, i)),
                  pl.BlockSpec((ND * D, ND), lambda i: (0, 0))],
        out_specs=pl.BlockSpec((ND * D, nb), lambda i: (0, i)),
        out_shape=jax.ShapeDtypeStruct((ND * D, B), jnp.float32),
    )(xt, wt)


def _sc_body(tbl_hbm, sidx_hbm, dtok_hbm, out_hbm,
             idxv, gbuf, dbuf, isem, gsemA, gsemB, ssem, dlsem, dsem):
    wid = lax.axis_index("s") * NC + lax.axis_index("c")

    def fire_gathers(f, par, gsem):
        goff = par * B
        slab = tbl_hbm.at[f * D + wid]

        def gblk(r8, c):
            for j in range(8):
                r = r8 * 8 + j
                pltpu.async_copy(slab.at[idxv.at[par, r]],
                                 gbuf.at[pl.ds(goff + r * 128, 128)], gsem)
            return c
        lax.fori_loop(0, RPF // 8, gblk, 0)

    def drain_gathers(par, gsem):
        pltpu.make_async_copy(tbl_hbm.at[0].at[pl.ds(0, B)],
                              gbuf.at[pl.ds(par * B, B)], gsem).wait()

    def drain_write(sem):
        pltpu.make_async_copy(gbuf.at[pl.ds(0, B)], out_hbm.at[0], sem).wait()

    def drain_idx():
        pltpu.make_async_copy(sidx_hbm.at[0], idxv.at[0], isem).wait()

    pltpu.sync_copy(sidx_hbm.at[0], idxv.at[0])
    fire_gathers(0, 0, gsemA)
    pltpu.async_copy(sidx_hbm.at[1], idxv.at[1], isem)

    def field_body(f, carry):
        par = lax.rem(f, 2)

        drain_idx()

        @pl.when(f >= 2)
        def _():
            drain_write(ssem)

        @pl.when(par == 0)
        def _():
            fire_gathers(f, 0, gsemA)
            drain_gathers(1, gsemB)

        @pl.when(par == 1)
        def _():
            fire_gathers(f, 1, gsemB)
            drain_gathers(0, gsemA)

        pltpu.async_copy(gbuf.at[pl.ds((1 - par) * B, B)],
                         out_hbm.at[(f - 1) * D + wid], ssem)

        @pl.when(f < F - 1)
        def _():
            pltpu.async_copy(sidx_hbm.at[f + 1], idxv.at[1 - par], isem)

        jd = f - 2

        @pl.when((jd >= 0) & (jd < ND))
        def _():
            @pl.when(jd >= 2)
            def _():
                drain_write(dsem)
            pltpu.async_copy(dtok_hbm.at[jd * D + wid],
                             dbuf.at[pl.ds(lax.rem(jd, 2) * B, B)], dlsem)

            @pl.when(jd >= 1)
            def _():
                drain_write(dlsem)
                pltpu.async_copy(dbuf.at[pl.ds(lax.rem(jd - 1, 2) * B, B)],
                                 out_hbm.at[(F + jd - 1) * D + wid], dsem)
        return carry
    lax.fori_loop(1, F, field_body, 0)

    drain_gathers(1, gsemB)
    pltpu.async_copy(gbuf.at[pl.ds(B, B)], out_hbm.at[(F - 1) * D + wid], ssem)
    drain_write(dlsem)
    pltpu.async_copy(dbuf.at[pl.ds(0, B)],
                     out_hbm.at[(F + ND - 1) * D + wid], dsem)
    drain_write(ssem)
    drain_write(ssem)
    drain_write(dsem)
    drain_write(dsem)


@jax.jit
def _tokenize(sparse_inputs, dense_inputs, tables, w):
    tbl_t = tables.transpose(0, 2, 1).reshape(F * D, V)
    sidx_t = sparse_inputs.T.reshape(F, RPF, 128)
    dtok_t = _dense_proj_t(dense_inputs.T, w.T)

    mesh = plsc.VectorSubcoreMesh(core_axis_name="c", subcore_axis_name="s",
                                  num_cores=NC, num_subcores=NS)
    sc = pl.kernel(
        _sc_body,
        out_type=jax.ShapeDtypeStruct((NF * D, B), jnp.float32),
        mesh=mesh,
        scratch_types=[
            pltpu.VMEM((2, RPF, 128), jnp.int32),
            pltpu.VMEM((2 * B,), jnp.float32),
            pltpu.VMEM((2 * B,), jnp.float32),
            pltpu.SemaphoreType.DMA,
            pltpu.SemaphoreType.DMA,
            pltpu.SemaphoreType.DMA,
            pltpu.SemaphoreType.DMA,
            pltpu.SemaphoreType.DMA,
            pltpu.SemaphoreType.DMA,
        ],
        compiler_params=pltpu.CompilerParams(use_tc_tiling_on_sc=False),
    )
    out = sc(tbl_t, sidx_t, dtok_t)
    return out.reshape(NF, D, B).transpose(2, 0, 1)


def kernel(sparse_inputs, dense_inputs, tables, W):
    return _tokenize(sparse_inputs, dense_inputs, tables, W)
